# baseline (speedup 1.0000x reference)
import functools

import jax
import jax.numpy as jnp
from jax import lax
from jax.experimental import pallas as pl
from jax.experimental.pallas import tpu as pltpu
from jax.experimental.pallas import tpu_sc as plsc

_VOCAB = 100000
_N_PREFIX = 100
_DIM = 64
_LANES = 16
_NC = 2
_NS = 16
_NW = _NC * _NS
_B = 16384
_L = 50
_CHUNK = 256
_SUB = 128
_NBUF = 2
_BQ = _B // _CHUNK
_DT = _DIM // 8
_PLANE = _CHUNK * 8


@functools.cache
def _make_gather(n_idx):
    n_chunks = n_idx // _CHUNK
    n_per_w = n_chunks // _NW
    mesh = plsc.VectorSubcoreMesh(core_axis_name="c", subcore_axis_name="s",
                                  num_cores=_NC, num_subcores=_NS)

    @functools.partial(
        pl.kernel,
        out_type=jax.ShapeDtypeStruct((n_idx * _DIM,), jnp.float32),
        mesh=mesh,
        compiler_params=pltpu.CompilerParams(use_tc_tiling_on_sc=False,
                                             needs_layout_passes=False),
        scratch_types=[
            pltpu.VMEM((_NBUF, _CHUNK), jnp.int32),
            pltpu.VMEM((_NBUF, _CHUNK), jnp.int32),
            pltpu.VMEM((_NBUF, _CHUNK, _DIM), jnp.float32),
            pltpu.VMEM((_NBUF, _DIM * _CHUNK), jnp.float32),
            pltpu.VMEM((_N_PREFIX, _DIM), jnp.float32),
            pltpu.SemaphoreType.DMA,
            pltpu.SemaphoreType.DMA,
            pltpu.SemaphoreType.DMA,
        ],
    )
    def gather_kernel(w_hbm, wn_hbm, idx_hbm, out_hbm,
                      idxo_v, idxc_v, rows_v, stage_v, wn_v,
                      gsem, ssem0, ssem1):
        wid = lax.axis_index("s") * _NC + lax.axis_index("c")
        ssems = (ssem0, ssem1)
        pltpu.sync_copy(wn_hbm, wn_v)

        def out_offsets(c):
            l = c // _BQ
            bq = c % _BQ
            p0 = l * _B + bq * _CHUNK
            o0 = (l * _DT * _BQ + bq) * _PLANE
            return p0, o0

        def do_chunk(g, b, first):
            c = wid * n_per_w + g
            p0, o0 = out_offsets(c)
            idxo = idxo_v.at[b]
            idxc = idxc_v.at[b]
            rows = rows_v.at[b]
            stage = stage_v.at[b]
            pltpu.sync_copy(idx_hbm.at[pl.ds(p0, _CHUNK)], idxo)

            def clamp_body(j, has_prefix):
                v = idxo[pl.ds(j * _LANES, _LANES)]
                m = v >= _VOCAB
                idxc[pl.ds(j * _LANES, _LANES)] = jnp.where(m, _VOCAB - 1, v)
                return has_prefix | jnp.any(m)

            has_prefix = lax.fori_loop(0, _CHUNK // _LANES, clamp_body,
                                       jnp.bool_(False))

            copies = [
                pltpu.async_copy(
                    w_hbm.at[idxc.at[pl.ds(k * _SUB, _SUB)]],
                    rows.at[pl.ds(k * _SUB, _SUB)],
                    gsem,
                )
                for k in range(_CHUNK // _SUB)
            ]
            for cp in copies:
                cp.wait()

            @pl.when(has_prefix)
            def _patch():
                def patch_slice(j, acc):
                    v = idxo[pl.ds(j * _LANES, _LANES)]
                    m = v >= _VOCAB

                    @pl.when(jnp.any(m))
                    def _do_patch():
                        e = jnp.where(m, v - _VOCAB, 0)
                        rows16 = j * _LANES + lax.iota(jnp.int32, _LANES)

                        def col_body(d, cc):
                            colv = jnp.full((_LANES,), d, jnp.int32)
                            vals = plsc.load_gather(wn_v, [e, colv], mask=m)
                            plsc.store_scatter(rows, [rows16, colv], vals,
                                               mask=m)
                            return cc

                        lax.fori_loop(0, _DIM, col_body, jnp.int32(0))

                    return acc

                lax.fori_loop(0, _CHUNK // _LANES, patch_slice, jnp.int32(0))

            @pl.when(jnp.logical_not(first))
            def _drain_prev():
                for dt in range(_DT):
                    pltpu.make_async_copy(
                        stage.at[pl.ds(dt * _PLANE, _PLANE)],
                        out_hbm.at[pl.ds(o0 + dt * _BQ * _PLANE, _PLANE)],
                        ssems[b]).wait()

            lanes16 = lax.iota(jnp.int32, _LANES)

            def tr_body(dt, acc):
                for bt in range(_CHUNK // _SUB):
                    for di in range(8):
                        colv = jnp.full((_LANES,), dt * 8 + di, jnp.int32)
                        base = dt * _PLANE + bt * (8 * _SUB) + di * _SUB
                        for bi in range(0, _SUB, _LANES):
                            rowv = bt * _SUB + bi + lanes16
                            vals = plsc.load_gather(rows, [rowv, colv])
                            stage[pl.ds(base + bi, _LANES)] = vals
                return acc

            lax.fori_loop(0, _DT, tr_body, jnp.int32(0))

            for dt in range(_DT):
                pltpu.async_copy(
                    stage.at[pl.ds(dt * _PLANE, _PLANE)],
                    out_hbm.at[pl.ds(o0 + dt * _BQ * _PLANE, _PLANE)],
                    ssems[b])

        def step_body(s, carry):
            for b in range(_NBUF):
                do_chunk(s * _NBUF + b, b, s < 1)
            return carry

        lax.fori_loop(0, n_per_w // _NBUF, step_body, jnp.int32(0))

        for b in range(_NBUF):
            g = (n_per_w // _NBUF - 1) * _NBUF + b
            _, o0 = out_offsets(wid * n_per_w + g)
            for dt in range(_DT):
                pltpu.make_async_copy(
                    stage_v.at[b].at[pl.ds(dt * _PLANE, _PLANE)],
                    out_hbm.at[pl.ds(o0 + dt * _BQ * _PLANE, _PLANE)],
                    ssems[b]).wait()

    return gather_kernel


def kernel(input, W, W_new):
    b, l = input.shape
    idx = input.T.reshape(-1).astype(jnp.int32)
    flat = _make_gather(idx.shape[0])(W, W_new, idx)
    out5 = flat.reshape(l, _DIM // 8, b // _SUB, 8, _SUB)
    return out5.transpose(2, 4, 0, 1, 3).reshape(b, l, _DIM)


# speedup vs baseline: 2.2029x; 2.2029x over previous
import functools

import jax
import jax.numpy as jnp
from jax import lax
from jax.experimental import pallas as pl
from jax.experimental.pallas import tpu as pltpu
from jax.experimental.pallas import tpu_sc as plsc

_VOCAB = 100000
_N_PREFIX = 100
_DIM = 64
_LANES = 16
_NC = 2
_NS = 16
_NW = _NC * _NS
_B = 16384
_L = 50
_CHUNK = 256
_SUB = 128
_NBUF = 2
_BQ = _B // _CHUNK
_DT = _DIM // 8
_NBT = _CHUNK // _SUB
_PITCH = _SUB + 1


@functools.cache
def _make_gather(n_idx):
    n_chunks = n_idx // _CHUNK
    n_per_w = n_chunks // _NW
    n_rows = n_idx * _DIM // _SUB
    mesh = plsc.VectorSubcoreMesh(core_axis_name="c", subcore_axis_name="s",
                                  num_cores=_NC, num_subcores=_NS)

    @functools.partial(
        pl.kernel,
        out_type=jax.ShapeDtypeStruct((n_rows, _SUB), jnp.float32),
        mesh=mesh,
        compiler_params=pltpu.CompilerParams(use_tc_tiling_on_sc=False,
                                             needs_layout_passes=False),
        scratch_types=[
            pltpu.VMEM((_NBUF, _CHUNK), jnp.int32),
            pltpu.VMEM((_NBUF, _CHUNK), jnp.int32),
            pltpu.VMEM((_NBUF, _CHUNK, _DIM), jnp.float32),
            pltpu.VMEM((_NBUF, _NBT * _DIM, _PITCH), jnp.float32),
            pltpu.VMEM((_N_PREFIX, _DIM), jnp.float32),
            pltpu.SemaphoreType.DMA,
            pltpu.SemaphoreType.DMA,
            pltpu.SemaphoreType.DMA,
        ],
    )
    def gather_kernel(w_hbm, wn_hbm, idx_hbm, out_hbm,
                      idxo_v, idxc_v, rows_v, stage_v, wn_v,
                      gsem, ssem0, ssem1):
        wid = lax.axis_index("s") * _NC + lax.axis_index("c")
        ssems = (ssem0, ssem1)
        pltpu.sync_copy(wn_hbm, wn_v)
        lanes16 = lax.iota(jnp.int32, _LANES)
        dg_vecs = [dg * _LANES + lanes16 for dg in range(_DIM // _LANES)]

        def offsets(c):
            l = c // _BQ
            bq = c % _BQ
            p0 = l * _B + bq * _CHUNK
            r0 = (l * _DT * (_B // _SUB) + bq * _NBT) * 8
            return p0, r0

        def store_dmas(stage, r0, sem):
            return [
                pltpu.make_async_copy(
                    stage.at[pl.ds(bt * _DIM + dt * 8, 8), pl.ds(0, _SUB)],
                    out_hbm.at[pl.ds(r0 + dt * (_B // _SUB) * 8 + bt * 8, 8)],
                    sem,
                )
                for dt in range(_DT)
                for bt in range(_NBT)
            ]

        def do_chunk(g, b, first):
            c = wid * n_per_w + g
            p0, r0 = offsets(c)
            idxo = idxo_v.at[b]
            idxc = idxc_v.at[b]
            rows = rows_v.at[b]
            stage = stage_v.at[b]
            pltpu.sync_copy(idx_hbm.at[pl.ds(p0, _CHUNK)], idxo)

            def clamp_body(j, has_prefix):
                v = idxo[pl.ds(j * _LANES, _LANES)]
                m = v >= _VOCAB
                idxc[pl.ds(j * _LANES, _LANES)] = jnp.where(m, _VOCAB - 1, v)
                return has_prefix | jnp.any(m)

            has_prefix = lax.fori_loop(0, _CHUNK // _LANES, clamp_body,
                                       jnp.bool_(False))

            copies = [
                pltpu.async_copy(
                    w_hbm.at[idxc.at[pl.ds(k * _SUB, _SUB)]],
                    rows.at[pl.ds(k * _SUB, _SUB)],
                    gsem,
                )
                for k in range(_NBT)
            ]
            for cp in copies:
                cp.wait()

            @pl.when(has_prefix)
            def _patch():
                def patch_slice(j, acc):
                    v = idxo[pl.ds(j * _LANES, _LANES)]
                    m = v >= _VOCAB

                    @pl.when(jnp.any(m))
                    def _do_patch():
                        e = jnp.where(m, v - _VOCAB, 0)
                        rows16 = j * _LANES + lanes16

                        def col_body(d, cc):
                            colv = jnp.full((_LANES,), d, jnp.int32)
                            vals = plsc.load_gather(wn_v, [e, colv], mask=m)
                            plsc.store_scatter(rows, [rows16, colv], vals,
                                               mask=m)
                            return cc

                        lax.fori_loop(0, _DIM, col_body, jnp.int32(0))

                    return acc

                lax.fori_loop(0, _CHUNK // _LANES, patch_slice, jnp.int32(0))

            @pl.when(jnp.logical_not(first))
            def _drain_prev():
                for cp in store_dmas(stage, r0, ssems[b]):
                    cp.wait()

            def tr_body(r, acc):
                bt = lax.shift_right_logical(r, 7)
                bi = lax.bitwise_and(r, _SUB - 1)
                biv = jnp.full((_LANES,), bi, jnp.int32)
                pbase = bt * _DIM
                for dg in range(_DIM // _LANES):
                    pv = pbase + dg_vecs[dg]
                    vals = rows[r, pl.ds(dg * _LANES, _LANES)]
                    plsc.store_scatter(stage, [pv, biv], vals)
                return acc

            lax.fori_loop(0, _CHUNK, tr_body, jnp.int32(0))

            for cp in store_dmas(stage, r0, ssems[b]):
                cp.start()

        def step_body(s, carry):
            for b in range(_NBUF):
                do_chunk(s * _NBUF + b, b, s < 1)
            return carry

        lax.fori_loop(0, n_per_w // _NBUF, step_body, jnp.int32(0))

        for b in range(_NBUF):
            g = (n_per_w // _NBUF - 1) * _NBUF + b
            _, r0 = offsets(wid * n_per_w + g)
            for cp in store_dmas(stage_v.at[b], r0, ssems[b]):
                cp.wait()

    return gather_kernel


def kernel(input, W, W_new):
    b, l = input.shape
    idx = input.T.reshape(-1).astype(jnp.int32)
    flat = _make_gather(idx.shape[0])(W, W_new, idx)
    out5 = flat.reshape(l, _DIM // 8, b // _SUB, 8, _SUB)
    return out5.transpose(2, 4, 0, 1, 3).reshape(b, l, _DIM)


# speedup vs baseline: 3.8114x; 1.7302x over previous
import functools

import jax
import jax.numpy as jnp
from jax import lax
from jax.experimental import pallas as pl
from jax.experimental.pallas import tpu as pltpu
from jax.experimental.pallas import tpu_sc as plsc

_VOCAB = 100000
_N_PREFIX = 100
_DIM = 64
_LANES = 16
_NC = 2
_NS = 16
_NW = _NC * _NS
_B = 16384
_L = 50
_CHUNK = 256
_SUB = 128
_NBUF = 2
_BQ = _B // _CHUNK
_DT = _DIM // 8
_NBT = _CHUNK // _SUB
_PITCH = _SUB + 1


@functools.cache
def _make_gather(n_idx):
    n_chunks = n_idx // _CHUNK
    n_per_w = n_chunks // _NW
    n_rows = n_idx * _DIM // _SUB
    mesh = plsc.VectorSubcoreMesh(core_axis_name="c", subcore_axis_name="s",
                                  num_cores=_NC, num_subcores=_NS)

    @functools.partial(
        pl.kernel,
        out_type=jax.ShapeDtypeStruct((n_rows, _SUB), jnp.float32),
        mesh=mesh,
        compiler_params=pltpu.CompilerParams(use_tc_tiling_on_sc=False,
                                             needs_layout_passes=False),
        scratch_types=[
            pltpu.VMEM((_NBUF, _CHUNK), jnp.int32),
            pltpu.VMEM((_NBUF, _CHUNK), jnp.int32),
            pltpu.VMEM((_NBUF, _CHUNK, _DIM), jnp.float32),
            pltpu.VMEM((_NBUF, _NBT * _DIM, _PITCH), jnp.float32),
            pltpu.VMEM((_N_PREFIX, _DIM), jnp.float32),
            pltpu.SemaphoreType.DMA,
            pltpu.SemaphoreType.DMA,
            pltpu.SemaphoreType.DMA,
        ],
    )
    def gather_kernel(w_hbm, wn_hbm, idx_hbm, out_hbm,
                      idxo_v, idxc_v, rows_v, stage_v, wn_v,
                      gsem, ssem0, ssem1):
        wid = lax.axis_index("s") * _NC + lax.axis_index("c")
        ssems = (ssem0, ssem1)
        pltpu.sync_copy(wn_hbm, wn_v)
        lanes16 = lax.iota(jnp.int32, _LANES)
        dg_vecs = [dg * _LANES + lanes16 for dg in range(_DIM // _LANES)]

        def offsets(c):
            l = c // _BQ
            bq = c % _BQ
            p0 = l * _B + bq * _CHUNK
            r0 = (l * _DT * (_B // _SUB) + bq * _NBT) * 8
            return p0, r0

        def store_dmas(stage, r0, sem):
            return [
                pltpu.make_async_copy(
                    stage.at[pl.ds(bt * _DIM + dt * 8, 8), pl.ds(0, _SUB)],
                    out_hbm.at[pl.ds(r0 + dt * (_B // _SUB) * 8 + bt * 8, 8)],
                    sem,
                )
                for dt in range(_DT)
                for bt in range(_NBT)
            ]

        def do_chunk(g, b, first):
            c = wid * n_per_w + g
            p0, r0 = offsets(c)
            idxo = idxo_v.at[b]
            idxc = idxc_v.at[b]
            rows = rows_v.at[b]
            stage = stage_v.at[b]
            pltpu.sync_copy(idx_hbm.at[pl.ds(p0, _CHUNK)], idxo)

            @plsc.parallel_loop(0, _CHUNK // _LANES, unroll=4,
                                carry=jnp.bool_(False))
            def has_prefix(j, acc):
                v = idxo[pl.ds(j * _LANES, _LANES)]
                m = v >= _VOCAB
                idxc[pl.ds(j * _LANES, _LANES)] = jnp.where(m, _VOCAB - 1, v)
                return acc | jnp.any(m)

            copies = [
                pltpu.async_copy(
                    w_hbm.at[idxc.at[pl.ds(k * _SUB, _SUB)]],
                    rows.at[pl.ds(k * _SUB, _SUB)],
                    gsem,
                )
                for k in range(_NBT)
            ]
            for cp in copies:
                cp.wait()

            @pl.when(has_prefix)
            def _patch():
                def patch_slice(j, acc):
                    v = idxo[pl.ds(j * _LANES, _LANES)]
                    m = v >= _VOCAB

                    @pl.when(jnp.any(m))
                    def _do_patch():
                        e = jnp.where(m, v - _VOCAB, 0)
                        rows16 = j * _LANES + lanes16

                        def col_body(d, cc):
                            colv = jnp.full((_LANES,), d, jnp.int32)
                            vals = plsc.load_gather(wn_v, [e, colv], mask=m)
                            plsc.store_scatter(rows, [rows16, colv], vals,
                                               mask=m)
                            return cc

                        lax.fori_loop(0, _DIM, col_body, jnp.int32(0))

                    return acc

                lax.fori_loop(0, _CHUNK // _LANES, patch_slice, jnp.int32(0))

            @pl.when(jnp.logical_not(first))
            def _drain_prev():
                for cp in store_dmas(stage, r0, ssems[b]):
                    cp.wait()

            @plsc.parallel_loop(0, _CHUNK, unroll=8)
            def _tr(r):
                bt = lax.shift_right_logical(r, 7)
                bi = lax.bitwise_and(r, _SUB - 1)
                biv = jnp.full((_LANES,), bi, jnp.int32)
                pbase = bt * _DIM
                for dg in range(_DIM // _LANES):
                    pv = pbase + dg_vecs[dg]
                    vals = rows[r, pl.ds(dg * _LANES, _LANES)]
                    plsc.store_scatter(stage, [pv, biv], vals)

            for cp in store_dmas(stage, r0, ssems[b]):
                cp.start()

        def step_body(s, carry):
            for b in range(_NBUF):
                do_chunk(s * _NBUF + b, b, s < 1)
            return carry

        lax.fori_loop(0, n_per_w // _NBUF, step_body, jnp.int32(0))

        for b in range(_NBUF):
            g = (n_per_w // _NBUF - 1) * _NBUF + b
            _, r0 = offsets(wid * n_per_w + g)
            for cp in store_dmas(stage_v.at[b], r0, ssems[b]):
                cp.wait()

    return gather_kernel


def kernel(input, W, W_new):
    b, l = input.shape
    idx = input.T.reshape(-1).astype(jnp.int32)
    flat = _make_gather(idx.shape[0])(W, W_new, idx)
    out5 = flat.reshape(l, _DIM // 8, b // _SUB, 8, _SUB)
    return out5.transpose(2, 4, 0, 1, 3).reshape(b, l, _DIM)
